# baseline (speedup 1.0000x reference)
import jax
import jax.numpy as jnp
from jax import lax
from jax.experimental import pallas as pl
from jax.experimental.pallas import tpu as pltpu
from jax.experimental.pallas import tpu_sc as plsc

NUM_CLASSES = 8192
EMD = 512
N = 4096
CT = 512
NT = NUM_CLASSES // CT
K_RM = int(0.25 * N)
EPS = 1e-06
RB = 512

NC = 2
NS = 16
NW = NC * NS
PASSES = 2
TPC = 128
SUMROWS = NUM_CLASSES
GB = 16
LCAP = N + 64


def _seg_body(rows_hbm, tgt_hbm, sums_hbm, cnts_hbm,
              tab, ctab, tgt_all, rid0, rel0, rid1, rel1, grow64,
              sem, wbsem, wbsem2):
    c = lax.axis_index("c")
    s = lax.axis_index("s")
    wid = s * NC + c
    iota16 = lax.iota(jnp.int32, 16)
    one16 = jnp.full((16,), 1.0, jnp.float32)
    zero16 = jnp.full((16,), 0.0, jnp.float32)
    izero16 = jnp.full((16,), 0, jnp.int32)

    pltpu.sync_copy(tgt_hbm, tgt_all)

    def prefill(q):
        rid0[pl.ds(q * 16, 16)] = izero16
        rid1[pl.ds(q * 16, 16)] = izero16
    pl.loop(0, LCAP // 16)(prefill)

    base0 = wid * TPC
    OFF1 = NW * TPC

    def compact(q, pos):
        pos0, pos1 = pos
        t = tgt_all[pl.ds(q * 16, 16)]
        rel = t - base0
        in0 = jnp.logical_and(rel >= 0, rel < TPC)
        in1 = jnp.logical_and(rel >= OFF1, rel < OFF1 + TPC)
        key = jnp.where(in0, 0, jnp.where(in1, 1, 2))
        _, perm = plsc.sort_key_val(key, iota16)
        rowc = jnp.take(q * 16 + iota16, perm)
        relc = jnp.take(rel, perm)
        cnt0 = plsc.all_reduce_population_count(in0)[0]
        cnt1 = plsc.all_reduce_population_count(in1)[0]
        rid0[pl.ds(pos0, 16)] = rowc
        rel0[pl.ds(pos0, 16)] = relc
        rot = jnp.bitwise_and(iota16 + cnt0, 15)
        rid1[pl.ds(pos1, 16)] = jnp.take(rowc, rot)
        rel1[pl.ds(pos1, 16)] = jnp.take(relc, rot) - OFF1
        return (pos0 + cnt0, pos1 + cnt1)

    m0, m1 = lax.fori_loop(0, N // 16, compact,
                           (jnp.int32(0), jnp.int32(0)))

    for p in range(PASSES):
        base_class = p * (NW * TPC) + wid * TPC
        ridl = rid0 if p == 0 else rid1
        rell = rel0 if p == 0 else rel1
        m = m0 if p == 0 else m1

        if p > 0:
            pltpu.make_async_copy(tab, sums_hbm.at[pl.ds(0, TPC)],
                                  wbsem).wait()
            pltpu.make_async_copy(ctab, cnts_hbm.at[pl.ds(0, TPC)],
                                  wbsem2).wait()

        def zrow_step(rw):
            ctab[rw, :] = zero16
            for k in range(EMD // 16):
                tab[rw, pl.ds(k * 16, 16)] = zero16
        pl.loop(0, TPC)(zrow_step)

        def accum(j, _):
            @pl.when(j % 3 == 0)
            def _():
                gbase = pl.multiple_of((j // 3) * 48, 48)
                pltpu.async_copy(rows_hbm.at[ridl.at[pl.ds(gbase, 48)]],
                                 grow64, sem).wait()
            rloc = (j % 3) * GB
            lane = j * GB + iota16
            relv = rell[pl.ds(j * GB, GB)]
            relv = jnp.where(lane < m, jnp.clip(relv, 0, TPC - 1), 0)
            validv = jnp.where(lane < m, 1.0, 0.0).astype(jnp.float32)
            for r in range(GB):
                rel_r = relv[r]
                v_r = validv[r]
                ctab[rel_r, :] = ctab[rel_r, :] + one16 * v_r
                for k in range(EMD // 16):
                    sl = pl.ds(k * 16, 16)
                    tab[rel_r, sl] = (tab[rel_r, sl]
                                      + grow64[rloc + r, sl] * v_r)
            return 0

        lax.fori_loop(0, (m + GB - 1) // GB, accum, 0)

        pltpu.async_copy(tab, sums_hbm.at[pl.ds(base_class, TPC)], wbsem)
        pltpu.async_copy(ctab, cnts_hbm.at[pl.ds(base_class, TPC)], wbsem2)

    pltpu.make_async_copy(tab, sums_hbm.at[pl.ds(0, TPC)], wbsem).wait()
    pltpu.make_async_copy(ctab, cnts_hbm.at[pl.ds(0, TPC)], wbsem2).wait()


def _segment_sums(inputs_row, trow_i32):
    mesh = plsc.VectorSubcoreMesh(core_axis_name="c", subcore_axis_name="s",
                                  num_cores=NC, num_subcores=NS)
    return pl.kernel(
        _seg_body,
        out_type=(jax.ShapeDtypeStruct((SUMROWS, EMD), jnp.float32),
                  jax.ShapeDtypeStruct((SUMROWS, 16), jnp.float32)),
        mesh=mesh,
        compiler_params=pltpu.CompilerParams(needs_layout_passes=False),
        scratch_types=[
            pltpu.VMEM((TPC, EMD), jnp.float32),
            pltpu.VMEM((TPC, 16), jnp.float32),
            pltpu.VMEM((N,), jnp.int32),
            pltpu.VMEM((LCAP,), jnp.int32),
            pltpu.VMEM((LCAP,), jnp.int32),
            pltpu.VMEM((LCAP,), jnp.int32),
            pltpu.VMEM((LCAP,), jnp.int32),
            pltpu.VMEM((48, EMD), jnp.float32),
            pltpu.SemaphoreType.DMA,
            pltpu.SemaphoreType.DMA,
            pltpu.SemaphoreType.DMA,
        ],
    )(inputs_row, trow_i32)


def _main_body(x_ref, tcol_ref, sums_ref, cnt_ref, cen_ref,
               out_ref, keep_ref, ok_ref, denom, num, flag):
    i = pl.program_id(0)
    base = i * CT
    colio = lax.broadcasted_iota(jnp.int32, (N, CT), 1)

    counts_t = cnt_ref[...][:, 0:1]
    filled_t = counts_t > 0.0
    cnew = jnp.where(filled_t, sums_ref[...] / jnp.maximum(counts_t, 1.0),
                     cen_ref[...])

    sims = lax.dot_general(x_ref[...], cnew.astype(jnp.bfloat16),
                           (((1,), (1,)), ((), ())),
                           preferred_element_type=jnp.float32)
    e = jnp.exp(sims)
    match = (colio == (tcol_ref[...] - base)).astype(jnp.float32)
    d_part = jnp.sum(e, axis=1, keepdims=True)
    n_part = jnp.sum(e * match, axis=1, keepdims=True)
    f_part = lax.dot_general(match, filled_t.astype(jnp.float32),
                             (((1,), (0,)), ((), ())),
                             preferred_element_type=jnp.float32)

    @pl.when(i == 0)
    def _():
        denom[...] = d_part
        num[...] = n_part
        flag[...] = f_part

    @pl.when(i > 0)
    def _():
        denom[...] += d_part
        num[...] += n_part
        flag[...] += f_part

    @pl.when(i == NT - 1)
    def _():
        c_all = jnp.where(flag[...] > 0.5,
                          num[...] / (denom[...] + EPS),
                          jnp.float32(1.0))
        out_ref[...] = c_all

        u = lax.bitcast_convert_type(c_all, jnp.int32)

        def bit_step(bb, res):
            cand = res | lax.shift_left(jnp.int32(1), 30 - bb)
            cnt = jnp.sum((u < cand).astype(jnp.float32))
            return jnp.where(cnt <= jnp.float32(K_RM - 1), cand, res)

        tbits = lax.fori_loop(0, 31, bit_step, jnp.int32(0))
        thr = lax.bitcast_convert_type(tbits, jnp.float32)
        maxc = jnp.max(c_all)
        common = jnp.logical_and(
            jnp.logical_and(thr == thr, thr != 1.0), maxc > thr)
        keep_ref[...] = jnp.where(common,
                                  (c_all > thr).astype(jnp.int32), 0)
        ok_ref[0, 0] = common.astype(jnp.int32)


def _rank_body(crow_ref, ccol_ref, keep_ref, rank_s, thr_s, maxc_s):
    p = pl.program_id(0)
    j = pl.program_id(1)
    cb = ccol_ref[...]

    @pl.when(jnp.logical_and(p == 0, j == 0))
    def _():
        thr_s[0, 0] = jnp.float32(0.0)
        maxc_s[0, 0] = jnp.float32(-jnp.inf)

    @pl.when(p == 0)
    def _():
        cr = crow_ref[...]
        jio = lax.broadcasted_iota(jnp.int32, (RB, N), 1)
        iio = lax.broadcasted_iota(jnp.int32, (RB, N), 0) + j * RB
        less = (cr < cb).astype(jnp.float32)
        tie = jnp.logical_and(cr == cb, jio < iio).astype(jnp.float32)
        rk = jnp.sum(less + tie, axis=1, keepdims=True)
        rank_s[pl.ds(j * RB, RB), :] = rk
        thr_s[0, 0] += jnp.sum(jnp.where(rk == jnp.float32(K_RM - 1), cb, 0.0))
        maxc_s[0, 0] = jnp.maximum(maxc_s[0, 0], jnp.max(cb))

    @pl.when(p == 1)
    def _():
        thr = thr_s[0, 0]
        valid = jnp.logical_and(thr == thr, thr != 1.0)
        anygt = maxc_s[0, 0] > thr
        rk = rank_s[pl.ds(j * RB, RB), :]
        gt_i = (cb > thr).astype(jnp.int32)
        fb_i = (rk >= jnp.float32(K_RM)).astype(jnp.int32)
        keep_ref[...] = jnp.where(jnp.logical_and(valid, anygt), gt_i, fb_i)


def kernel(inputs_col, targets_col, inputs_row, target_row, center):
    tcol = targets_col.astype(jnp.int32).reshape(N, 1)
    trow = target_row.astype(jnp.int32)

    sums, counts = _segment_sums(inputs_row, trow)

    c_col, keep_common, okflag = pl.pallas_call(
        _main_body,
        grid=(NT,),
        in_specs=[
            pl.BlockSpec((N, EMD), lambda i: (0, 0)),
            pl.BlockSpec((N, 1), lambda i: (0, 0)),
            pl.BlockSpec((CT, EMD), lambda i: (i, 0)),
            pl.BlockSpec((CT, 16), lambda i: (i, 0)),
            pl.BlockSpec((CT, EMD), lambda i: (i, 0)),
        ],
        out_specs=[
            pl.BlockSpec((N, 1), lambda i: (0, 0)),
            pl.BlockSpec((N, 1), lambda i: (0, 0)),
            pl.BlockSpec(memory_space=pltpu.SMEM),
        ],
        out_shape=[
            jax.ShapeDtypeStruct((N, 1), jnp.float32),
            jax.ShapeDtypeStruct((N, 1), jnp.int32),
            jax.ShapeDtypeStruct((1, 1), jnp.int32),
        ],
        scratch_shapes=[
            pltpu.VMEM((N, 1), jnp.float32),
            pltpu.VMEM((N, 1), jnp.float32),
            pltpu.VMEM((N, 1), jnp.float32),
        ],
    )(inputs_col.astype(jnp.bfloat16), tcol, sums, counts, center)

    def _rare_path(c):
        return pl.pallas_call(
            _rank_body,
            grid=(2, N // RB),
            in_specs=[
                pl.BlockSpec((1, N), lambda p, j: (0, 0)),
                pl.BlockSpec((RB, 1), lambda p, j: (j, 0)),
            ],
            out_specs=pl.BlockSpec((RB, 1), lambda p, j: (p * j, 0)),
            out_shape=jax.ShapeDtypeStruct((N, 1), jnp.int32),
            scratch_shapes=[
                pltpu.VMEM((N, 1), jnp.float32),
                pltpu.SMEM((1, 1), jnp.float32),
                pltpu.SMEM((1, 1), jnp.float32),
            ],
        )(c.reshape(1, N), c)

    keep_i = lax.cond(okflag[0, 0] != 0,
                      lambda c: keep_common,
                      _rare_path,
                      c_col)

    return (c_col.reshape(N), keep_i.reshape(N).astype(bool))


# speedup vs baseline: 1.0716x; 1.0716x over previous
import jax
import jax.numpy as jnp
from jax import lax
from jax.experimental import pallas as pl
from jax.experimental.pallas import tpu as pltpu
from jax.experimental.pallas import tpu_sc as plsc

NUM_CLASSES = 8192
EMD = 512
N = 4096
CT = 512
NT = NUM_CLASSES // CT
K_RM = int(0.25 * N)
EPS = 1e-06
RB = 512

NC = 2
NS = 16
NW = NC * NS
PASSES = 2
TPC = 128
SUMROWS = NUM_CLASSES
GB = 16
LCAP = N + 64


def _seg_body(rows_hbm, tgt_hbm, sums_hbm, cnts_hbm,
              tab, ctab, tgt_all, rowids, rels, grow_buf,
              sem, wbsem, wbsem2):
    c = lax.axis_index("c")
    s = lax.axis_index("s")
    wid = s * NC + c
    iota16 = lax.iota(jnp.int32, 16)
    one16 = jnp.full((16,), 1.0, jnp.float32)
    zero16 = jnp.full((16,), 0.0, jnp.float32)

    pltpu.sync_copy(tgt_hbm, tgt_all)

    for p in range(PASSES):
        base_class = p * (NW * TPC) + wid * TPC

        def compact(q, pos):
            t = tgt_all[pl.ds(q * 16, 16)]
            rel = t - base_class
            ok = jnp.logical_and(rel >= 0, rel < TPC)
            key = jnp.where(ok, 0, 1)
            _, perm = plsc.sort_key_val(key, iota16)
            rowids[pl.ds(pos, 16)] = jnp.take(q * 16 + iota16, perm)
            rels[pl.ds(pos, 16)] = jnp.take(rel, perm)
            cnt = plsc.all_reduce_population_count(ok)
            return pos + cnt[0]

        m = lax.fori_loop(0, N // 16, compact, jnp.int32(0))

        if p > 0:
            pltpu.make_async_copy(tab, sums_hbm.at[pl.ds(0, TPC)],
                                  wbsem).wait()
            pltpu.make_async_copy(ctab, cnts_hbm.at[pl.ds(0, TPC)],
                                  wbsem2).wait()

        def zrow_step(rw):
            ctab[rw, :] = zero16
            for k in range(EMD // 16):
                tab[rw, pl.ds(k * 16, 16)] = zero16
        pl.loop(0, TPC)(zrow_step)

        def accum(j, _):
            lane = j * GB + iota16
            ivec = rowids[pl.ds(j * GB, GB)]
            ivec = jnp.where(lane < m, jnp.clip(ivec, 0, N - 1), 0)
            relv = rels[pl.ds(j * GB, GB)]
            relv = jnp.where(lane < m, jnp.clip(relv, 0, TPC - 1), 0)
            validv = jnp.where(lane < m, 1.0, 0.0).astype(jnp.float32)
            pltpu.async_copy(rows_hbm.at[ivec], grow_buf, sem).wait()
            for r in range(GB):
                rel_r = relv[r]
                v_r = validv[r]
                ctab[rel_r, :] = ctab[rel_r, :] + one16 * v_r
                for k in range(EMD // 16):
                    sl = pl.ds(k * 16, 16)
                    tab[rel_r, sl] = tab[rel_r, sl] + grow_buf[r, sl] * v_r
            return 0

        lax.fori_loop(0, (m + GB - 1) // GB, accum, 0)

        pltpu.async_copy(tab, sums_hbm.at[pl.ds(base_class, TPC)], wbsem)
        pltpu.async_copy(ctab, cnts_hbm.at[pl.ds(base_class, TPC)], wbsem2)

    pltpu.make_async_copy(tab, sums_hbm.at[pl.ds(0, TPC)], wbsem).wait()
    pltpu.make_async_copy(ctab, cnts_hbm.at[pl.ds(0, TPC)], wbsem2).wait()


def _segment_sums(inputs_row, trow_i32):
    mesh = plsc.VectorSubcoreMesh(core_axis_name="c", subcore_axis_name="s",
                                  num_cores=NC, num_subcores=NS)
    return pl.kernel(
        _seg_body,
        out_type=(jax.ShapeDtypeStruct((SUMROWS, EMD), jnp.float32),
                  jax.ShapeDtypeStruct((SUMROWS, 16), jnp.float32)),
        mesh=mesh,
        compiler_params=pltpu.CompilerParams(needs_layout_passes=False),
        scratch_types=[
            pltpu.VMEM((TPC, EMD), jnp.float32),
            pltpu.VMEM((TPC, 16), jnp.float32),
            pltpu.VMEM((N,), jnp.int32),
            pltpu.VMEM((N + 16,), jnp.int32),
            pltpu.VMEM((N + 16,), jnp.int32),
            pltpu.VMEM((GB, EMD), jnp.float32),
            pltpu.SemaphoreType.DMA,
            pltpu.SemaphoreType.DMA,
            pltpu.SemaphoreType.DMA,
        ],
    )(inputs_row, trow_i32)


def _main_body(x_ref, tcol_ref, sums_ref, cnt_ref, cen_ref,
               out_ref, keep_ref, ok_ref, denom, num, flag):
    i = pl.program_id(0)
    base = i * CT
    colio = lax.broadcasted_iota(jnp.int32, (N, CT), 1)

    counts_t = cnt_ref[...][:, 0:1]
    filled_t = counts_t > 0.0
    cnew = jnp.where(filled_t, sums_ref[...] / jnp.maximum(counts_t, 1.0),
                     cen_ref[...])

    sims = lax.dot_general(x_ref[...], cnew.astype(jnp.bfloat16),
                           (((1,), (1,)), ((), ())),
                           preferred_element_type=jnp.float32)
    e = jnp.exp(sims)
    match = (colio == (tcol_ref[...] - base)).astype(jnp.float32)
    d_part = jnp.sum(e, axis=1, keepdims=True)
    n_part = jnp.sum(e * match, axis=1, keepdims=True)
    f_part = lax.dot_general(match, filled_t.astype(jnp.float32),
                             (((1,), (0,)), ((), ())),
                             preferred_element_type=jnp.float32)

    @pl.when(i == 0)
    def _():
        denom[...] = d_part
        num[...] = n_part
        flag[...] = f_part

    @pl.when(i > 0)
    def _():
        denom[...] += d_part
        num[...] += n_part
        flag[...] += f_part

    @pl.when(i == NT - 1)
    def _():
        c_all = jnp.where(flag[...] > 0.5,
                          num[...] / (denom[...] + EPS),
                          jnp.float32(1.0))
        out_ref[...] = c_all

        u = lax.bitcast_convert_type(c_all, jnp.int32)

        def bit_step(bb, res):
            cand = res | lax.shift_left(jnp.int32(1), 30 - bb)
            cnt = jnp.sum((u < cand).astype(jnp.float32))
            return jnp.where(cnt <= jnp.float32(K_RM - 1), cand, res)

        tbits = lax.fori_loop(0, 31, bit_step, jnp.int32(0))
        thr = lax.bitcast_convert_type(tbits, jnp.float32)
        maxc = jnp.max(c_all)
        common = jnp.logical_and(
            jnp.logical_and(thr == thr, thr != 1.0), maxc > thr)
        keep_ref[...] = jnp.where(common,
                                  (c_all > thr).astype(jnp.int32), 0)
        ok_ref[0, 0] = common.astype(jnp.int32)


def _rank_body(crow_ref, ccol_ref, keep_ref, rank_s, thr_s, maxc_s):
    p = pl.program_id(0)
    j = pl.program_id(1)
    cb = ccol_ref[...]

    @pl.when(jnp.logical_and(p == 0, j == 0))
    def _():
        thr_s[0, 0] = jnp.float32(0.0)
        maxc_s[0, 0] = jnp.float32(-jnp.inf)

    @pl.when(p == 0)
    def _():
        cr = crow_ref[...]
        jio = lax.broadcasted_iota(jnp.int32, (RB, N), 1)
        iio = lax.broadcasted_iota(jnp.int32, (RB, N), 0) + j * RB
        less = (cr < cb).astype(jnp.float32)
        tie = jnp.logical_and(cr == cb, jio < iio).astype(jnp.float32)
        rk = jnp.sum(less + tie, axis=1, keepdims=True)
        rank_s[pl.ds(j * RB, RB), :] = rk
        thr_s[0, 0] += jnp.sum(jnp.where(rk == jnp.float32(K_RM - 1), cb, 0.0))
        maxc_s[0, 0] = jnp.maximum(maxc_s[0, 0], jnp.max(cb))

    @pl.when(p == 1)
    def _():
        thr = thr_s[0, 0]
        valid = jnp.logical_and(thr == thr, thr != 1.0)
        anygt = maxc_s[0, 0] > thr
        rk = rank_s[pl.ds(j * RB, RB), :]
        gt_i = (cb > thr).astype(jnp.int32)
        fb_i = (rk >= jnp.float32(K_RM)).astype(jnp.int32)
        keep_ref[...] = jnp.where(jnp.logical_and(valid, anygt), gt_i, fb_i)


def kernel(inputs_col, targets_col, inputs_row, target_row, center):
    tcol = targets_col.astype(jnp.int32).reshape(N, 1)
    trow = target_row.astype(jnp.int32)

    sums, counts = _segment_sums(inputs_row, trow)

    c_col, keep_common, okflag = pl.pallas_call(
        _main_body,
        grid=(NT,),
        in_specs=[
            pl.BlockSpec((N, EMD), lambda i: (0, 0)),
            pl.BlockSpec((N, 1), lambda i: (0, 0)),
            pl.BlockSpec((CT, EMD), lambda i: (i, 0)),
            pl.BlockSpec((CT, 16), lambda i: (i, 0)),
            pl.BlockSpec((CT, EMD), lambda i: (i, 0)),
        ],
        out_specs=[
            pl.BlockSpec((N, 1), lambda i: (0, 0)),
            pl.BlockSpec((N, 1), lambda i: (0, 0)),
            pl.BlockSpec(memory_space=pltpu.SMEM),
        ],
        out_shape=[
            jax.ShapeDtypeStruct((N, 1), jnp.float32),
            jax.ShapeDtypeStruct((N, 1), jnp.int32),
            jax.ShapeDtypeStruct((1, 1), jnp.int32),
        ],
        scratch_shapes=[
            pltpu.VMEM((N, 1), jnp.float32),
            pltpu.VMEM((N, 1), jnp.float32),
            pltpu.VMEM((N, 1), jnp.float32),
        ],
    )(inputs_col.astype(jnp.bfloat16), tcol, sums, counts, center)

    def _rare_path(c):
        return pl.pallas_call(
            _rank_body,
            grid=(2, N // RB),
            in_specs=[
                pl.BlockSpec((1, N), lambda p, j: (0, 0)),
                pl.BlockSpec((RB, 1), lambda p, j: (j, 0)),
            ],
            out_specs=pl.BlockSpec((RB, 1), lambda p, j: (p * j, 0)),
            out_shape=jax.ShapeDtypeStruct((N, 1), jnp.int32),
            scratch_shapes=[
                pltpu.VMEM((N, 1), jnp.float32),
                pltpu.SMEM((1, 1), jnp.float32),
                pltpu.SMEM((1, 1), jnp.float32),
            ],
        )(c.reshape(1, N), c)

    keep_i = lax.cond(okflag[0, 0] != 0,
                      lambda c: keep_common,
                      _rare_path,
                      c_col)

    return (c_col.reshape(N), keep_i.reshape(N).astype(bool))


# speedup vs baseline: 1.0785x; 1.0065x over previous
import jax
import jax.numpy as jnp
from jax import lax
from jax.experimental import pallas as pl
from jax.experimental.pallas import tpu as pltpu
from jax.experimental.pallas import tpu_sc as plsc

NUM_CLASSES = 8192
EMD = 512
N = 4096
CT = 1024
NT = NUM_CLASSES // CT
K_RM = int(0.25 * N)
EPS = 1e-06
RB = 512

NC = 2
NS = 16
NW = NC * NS
PASSES = 2
TPC = 128
SUMROWS = NUM_CLASSES
GB = 16
LCAP = N + 64


def _seg_body(rows_hbm, tgt_hbm, sums_hbm, cnts_hbm,
              tab, ctab, tgt_all, rowids, rels, grow_buf,
              sem, wbsem, wbsem2):
    c = lax.axis_index("c")
    s = lax.axis_index("s")
    wid = s * NC + c
    iota16 = lax.iota(jnp.int32, 16)
    one16 = jnp.full((16,), 1.0, jnp.float32)
    zero16 = jnp.full((16,), 0.0, jnp.float32)

    pltpu.sync_copy(tgt_hbm, tgt_all)

    for p in range(PASSES):
        base_class = p * (NW * TPC) + wid * TPC

        def compact(q, pos):
            t = tgt_all[pl.ds(q * 16, 16)]
            rel = t - base_class
            ok = jnp.logical_and(rel >= 0, rel < TPC)
            key = jnp.where(ok, 0, 1)
            _, perm = plsc.sort_key_val(key, iota16)
            rowids[pl.ds(pos, 16)] = jnp.take(q * 16 + iota16, perm)
            rels[pl.ds(pos, 16)] = jnp.take(rel, perm)
            cnt = plsc.all_reduce_population_count(ok)
            return pos + cnt[0]

        m = lax.fori_loop(0, N // 16, compact, jnp.int32(0))

        if p > 0:
            pltpu.make_async_copy(tab, sums_hbm.at[pl.ds(0, TPC)],
                                  wbsem).wait()
            pltpu.make_async_copy(ctab, cnts_hbm.at[pl.ds(0, TPC)],
                                  wbsem2).wait()

        def zrow_step(rw):
            ctab[rw, :] = zero16
            for k in range(EMD // 16):
                tab[rw, pl.ds(k * 16, 16)] = zero16
        pl.loop(0, TPC)(zrow_step)

        def accum(j, _):
            lane = j * GB + iota16
            ivec = rowids[pl.ds(j * GB, GB)]
            ivec = jnp.where(lane < m, jnp.clip(ivec, 0, N - 1), 0)
            relv = rels[pl.ds(j * GB, GB)]
            relv = jnp.where(lane < m, jnp.clip(relv, 0, TPC - 1), 0)
            validv = jnp.where(lane < m, 1.0, 0.0).astype(jnp.float32)
            pltpu.async_copy(rows_hbm.at[ivec], grow_buf, sem).wait()
            for r in range(GB):
                rel_r = relv[r]
                v_r = validv[r]
                ctab[rel_r, :] = ctab[rel_r, :] + one16 * v_r
                for k in range(EMD // 16):
                    sl = pl.ds(k * 16, 16)
                    tab[rel_r, sl] = tab[rel_r, sl] + grow_buf[r, sl] * v_r
            return 0

        lax.fori_loop(0, (m + GB - 1) // GB, accum, 0)

        pltpu.async_copy(tab, sums_hbm.at[pl.ds(base_class, TPC)], wbsem)
        pltpu.async_copy(ctab, cnts_hbm.at[pl.ds(base_class, TPC)], wbsem2)

    pltpu.make_async_copy(tab, sums_hbm.at[pl.ds(0, TPC)], wbsem).wait()
    pltpu.make_async_copy(ctab, cnts_hbm.at[pl.ds(0, TPC)], wbsem2).wait()


def _segment_sums(inputs_row, trow_i32):
    mesh = plsc.VectorSubcoreMesh(core_axis_name="c", subcore_axis_name="s",
                                  num_cores=NC, num_subcores=NS)
    return pl.kernel(
        _seg_body,
        out_type=(jax.ShapeDtypeStruct((SUMROWS, EMD), jnp.float32),
                  jax.ShapeDtypeStruct((SUMROWS, 16), jnp.float32)),
        mesh=mesh,
        compiler_params=pltpu.CompilerParams(needs_layout_passes=False),
        scratch_types=[
            pltpu.VMEM((TPC, EMD), jnp.float32),
            pltpu.VMEM((TPC, 16), jnp.float32),
            pltpu.VMEM((N,), jnp.int32),
            pltpu.VMEM((N + 16,), jnp.int32),
            pltpu.VMEM((N + 16,), jnp.int32),
            pltpu.VMEM((GB, EMD), jnp.float32),
            pltpu.SemaphoreType.DMA,
            pltpu.SemaphoreType.DMA,
            pltpu.SemaphoreType.DMA,
        ],
    )(inputs_row, trow_i32)


def _main_body(x_ref, tcol_ref, sums_ref, cnt_ref, cen_ref,
               out_ref, keep_ref, ok_ref, denom, num, flag):
    i = pl.program_id(0)
    base = i * CT
    colio = lax.broadcasted_iota(jnp.int32, (N, CT), 1)

    counts_t = cnt_ref[...][:, 0:1]
    filled_t = counts_t > 0.0
    cnew = jnp.where(filled_t, sums_ref[...] / jnp.maximum(counts_t, 1.0),
                     cen_ref[...])

    sims = lax.dot_general(x_ref[...], cnew.astype(jnp.bfloat16),
                           (((1,), (1,)), ((), ())),
                           preferred_element_type=jnp.float32)
    e = jnp.exp(sims)
    match = (colio == (tcol_ref[...] - base)).astype(jnp.float32)
    d_part = jnp.sum(e, axis=1, keepdims=True)
    n_part = jnp.sum(e * match, axis=1, keepdims=True)
    f_part = lax.dot_general(match, filled_t.astype(jnp.float32),
                             (((1,), (0,)), ((), ())),
                             preferred_element_type=jnp.float32)

    @pl.when(i == 0)
    def _():
        denom[...] = d_part
        num[...] = n_part
        flag[...] = f_part

    @pl.when(i > 0)
    def _():
        denom[...] += d_part
        num[...] += n_part
        flag[...] += f_part

    @pl.when(i == NT - 1)
    def _():
        c_all = jnp.where(flag[...] > 0.5,
                          num[...] / (denom[...] + EPS),
                          jnp.float32(1.0))
        out_ref[...] = c_all

        u = lax.bitcast_convert_type(c_all, jnp.int32)

        def bit_step(bb, res):
            cand = res | lax.shift_left(jnp.int32(1), 30 - bb)
            cnt = jnp.sum((u < cand).astype(jnp.float32))
            return jnp.where(cnt <= jnp.float32(K_RM - 1), cand, res)

        tbits = lax.fori_loop(0, 31, bit_step, jnp.int32(0))
        thr = lax.bitcast_convert_type(tbits, jnp.float32)
        maxc = jnp.max(c_all)
        common = jnp.logical_and(
            jnp.logical_and(thr == thr, thr != 1.0), maxc > thr)
        keep_ref[...] = jnp.where(common,
                                  (c_all > thr).astype(jnp.int32), 0)
        ok_ref[0, 0] = common.astype(jnp.int32)


def _rank_body(crow_ref, ccol_ref, keep_ref, rank_s, thr_s, maxc_s):
    p = pl.program_id(0)
    j = pl.program_id(1)
    cb = ccol_ref[...]

    @pl.when(jnp.logical_and(p == 0, j == 0))
    def _():
        thr_s[0, 0] = jnp.float32(0.0)
        maxc_s[0, 0] = jnp.float32(-jnp.inf)

    @pl.when(p == 0)
    def _():
        cr = crow_ref[...]
        jio = lax.broadcasted_iota(jnp.int32, (RB, N), 1)
        iio = lax.broadcasted_iota(jnp.int32, (RB, N), 0) + j * RB
        less = (cr < cb).astype(jnp.float32)
        tie = jnp.logical_and(cr == cb, jio < iio).astype(jnp.float32)
        rk = jnp.sum(less + tie, axis=1, keepdims=True)
        rank_s[pl.ds(j * RB, RB), :] = rk
        thr_s[0, 0] += jnp.sum(jnp.where(rk == jnp.float32(K_RM - 1), cb, 0.0))
        maxc_s[0, 0] = jnp.maximum(maxc_s[0, 0], jnp.max(cb))

    @pl.when(p == 1)
    def _():
        thr = thr_s[0, 0]
        valid = jnp.logical_and(thr == thr, thr != 1.0)
        anygt = maxc_s[0, 0] > thr
        rk = rank_s[pl.ds(j * RB, RB), :]
        gt_i = (cb > thr).astype(jnp.int32)
        fb_i = (rk >= jnp.float32(K_RM)).astype(jnp.int32)
        keep_ref[...] = jnp.where(jnp.logical_and(valid, anygt), gt_i, fb_i)


def kernel(inputs_col, targets_col, inputs_row, target_row, center):
    tcol = targets_col.astype(jnp.int32).reshape(N, 1)
    trow = target_row.astype(jnp.int32)

    sums, counts = _segment_sums(inputs_row, trow)

    c_col, keep_common, okflag = pl.pallas_call(
        _main_body,
        grid=(NT,),
        in_specs=[
            pl.BlockSpec((N, EMD), lambda i: (0, 0)),
            pl.BlockSpec((N, 1), lambda i: (0, 0)),
            pl.BlockSpec((CT, EMD), lambda i: (i, 0)),
            pl.BlockSpec((CT, 16), lambda i: (i, 0)),
            pl.BlockSpec((CT, EMD), lambda i: (i, 0)),
        ],
        out_specs=[
            pl.BlockSpec((N, 1), lambda i: (0, 0)),
            pl.BlockSpec((N, 1), lambda i: (0, 0)),
            pl.BlockSpec(memory_space=pltpu.SMEM),
        ],
        out_shape=[
            jax.ShapeDtypeStruct((N, 1), jnp.float32),
            jax.ShapeDtypeStruct((N, 1), jnp.int32),
            jax.ShapeDtypeStruct((1, 1), jnp.int32),
        ],
        scratch_shapes=[
            pltpu.VMEM((N, 1), jnp.float32),
            pltpu.VMEM((N, 1), jnp.float32),
            pltpu.VMEM((N, 1), jnp.float32),
        ],
    )(inputs_col.astype(jnp.bfloat16), tcol, sums, counts, center)

    def _rare_path(c):
        return pl.pallas_call(
            _rank_body,
            grid=(2, N // RB),
            in_specs=[
                pl.BlockSpec((1, N), lambda p, j: (0, 0)),
                pl.BlockSpec((RB, 1), lambda p, j: (j, 0)),
            ],
            out_specs=pl.BlockSpec((RB, 1), lambda p, j: (p * j, 0)),
            out_shape=jax.ShapeDtypeStruct((N, 1), jnp.int32),
            scratch_shapes=[
                pltpu.VMEM((N, 1), jnp.float32),
                pltpu.SMEM((1, 1), jnp.float32),
                pltpu.SMEM((1, 1), jnp.float32),
            ],
        )(c.reshape(1, N), c)

    keep_i = lax.cond(okflag[0, 0] != 0,
                      lambda c: keep_common,
                      _rare_path,
                      c_col)

    return (c_col.reshape(N), keep_i.reshape(N).astype(bool))
